# baseline (speedup 1.0000x reference)
import functools

import jax
import jax.numpy as jnp
from jax import lax
from jax.experimental import pallas as pl
from jax.experimental.pallas import tpu as pltpu
from jax.experimental.pallas import tpu_sc as plsc

STATE = 128
VOCAB = 100000
QALL = 300
B = 1024
L = 50
NWORDS = B * L

VC = 4096
NCHUNK = -(-VOCAB // VC)

_NEG = -1e30


def _lse_body(pred_ref, wt_ref, z_ref, lse_ref, m_ref, s_ref):
    i = pl.program_id(0)
    logits = lax.dot_general(pred_ref[...], wt_ref[...],
                             (((1,), (1,)), ((), ())),
                             preferred_element_type=jnp.float32)

    @pl.when(i == 0)
    def _():
        m_ref[...] = jnp.full((QALL, 1), _NEG, jnp.float32)
        s_ref[...] = jnp.zeros((QALL, 1), jnp.float32)

    def update(lm):
        m_old = m_ref[...]
        s_old = s_ref[...]
        m_new = jnp.maximum(m_old, jnp.max(lm, axis=1, keepdims=True))
        e = jnp.exp(lm - m_new)
        sc = jnp.dot(e, z_ref[...], preferred_element_type=jnp.float32)
        s_new = s_old * jnp.exp(m_old - m_new) + sc
        m_ref[...] = m_new
        s_ref[...] = s_new
        return m_new, s_new

    @pl.when(i < NCHUNK - 1)
    def _():
        update(logits)

    @pl.when(i == NCHUNK - 1)
    def _():
        col = i * VC + lax.broadcasted_iota(jnp.int32, (1, VC), 1)
        m_new, s_new = update(jnp.where(col < VOCAB, logits, _NEG))
        lse_ref[...] = m_new + jnp.log(s_new)


def _lse(predcat_emb, emit_wt, z2):
    return pl.pallas_call(
        _lse_body,
        grid=(NCHUNK,),
        in_specs=[
            pl.BlockSpec((QALL, STATE), lambda i: (0, 0)),
            pl.BlockSpec((VC, STATE), lambda i: (i, 0)),
            pl.BlockSpec((VC, 1), lambda i: (i, 0)),
        ],
        out_specs=pl.BlockSpec((QALL, 1), lambda i: (0, 0)),
        out_shape=jax.ShapeDtypeStruct((QALL, 1), jnp.float32),
        scratch_shapes=[
            pltpu.VMEM((QALL, 1), jnp.float32),
            pltpu.VMEM((QALL, 1), jnp.float32),
        ],
    )(predcat_emb, emit_wt, z2)


def _sc_gather(wt, emit_b, words_flat):
    info = plsc.get_sparse_core_info()
    nc, ns = info.num_cores, info.num_subcores
    nw = nc * ns
    b_per_w = NWORDS // nw
    nchunk = 4
    ch = b_per_w // nchunk

    mesh = plsc.VectorSubcoreMesh(core_axis_name="c", subcore_axis_name="s")

    @functools.partial(
        pl.kernel, mesh=mesh,
        out_type=[
            jax.ShapeDtypeStruct((NWORDS, STATE), jnp.float32),
            jax.ShapeDtypeStruct((NWORDS,), jnp.float32),
        ],
        scratch_types=[
            pltpu.VMEM((b_per_w,), jnp.int32),
            pltpu.VMEM((ch, STATE), jnp.float32),
            pltpu.VMEM((ch, STATE), jnp.float32),
            pltpu.VMEM((ch,), jnp.float32),
            pltpu.VMEM((ch,), jnp.float32),
            pltpu.SemaphoreType.DMA,
            pltpu.SemaphoreType.DMA,
        ],
    )
    def k(wt_hbm, b_hbm, words_hbm, g_hbm, bv_hbm, idx_v, rows_v0, rows_v1,
          brow_v0, brow_v1, sem_r, sem_b):
        wid = lax.axis_index("s") * nc + lax.axis_index("c")
        base = wid * b_per_w
        rows_v = (rows_v0, rows_v1)
        brow_v = (brow_v0, brow_v1)
        pltpu.sync_copy(words_hbm.at[pl.ds(base, b_per_w)], idx_v)

        def fire(c, slot):
            idx_c = idx_v.at[pl.ds(c * ch, ch)]
            pltpu.async_copy(wt_hbm.at[idx_c], rows_v[slot], sem_r)
            pltpu.async_copy(b_hbm.at[idx_c], brow_v[slot], sem_b)

        def drain(c, slot):
            pltpu.make_async_copy(
                wt_hbm.at[idx_v.at[pl.ds(c * ch, ch)]], rows_v[slot],
                sem_r).wait()
            pltpu.make_async_copy(
                b_hbm.at[idx_v.at[pl.ds(c * ch, ch)]], brow_v[slot],
                sem_b).wait()
            pltpu.sync_copy(rows_v[slot],
                            g_hbm.at[pl.ds(base + c * ch, ch)])
            pltpu.sync_copy(brow_v[slot],
                            bv_hbm.at[pl.ds(base + c * ch, ch)])

        fire(0, 0)
        for c in range(nchunk):
            if c + 1 < nchunk:
                fire(c + 1, (c + 1) % 2)
            drain(c, c % 2)

    return k(wt, emit_b, words_flat)


def _log_softmax_rows(x):
    m = jnp.max(x, axis=1, keepdims=True)
    return x - m - jnp.log(jnp.sum(jnp.exp(x - m), axis=1, keepdims=True))


def _emit_body(g_ref, bv_ref, predb_ref, pred_ref, lse_ref,
               root_W_ref, root_b_ref, rule_W_ref, rule_b_ref,
               op_W_ref, op_b_ref, s_in_W_ref, s_in_b_ref,
               r1_W1_ref, r1_b1_ref, r1_W2_ref, r1_b2_ref,
               r2_W1_ref, r2_b1_ref, r2_W2_ref, r2_b2_ref,
               s_out_W_ref, s_out_b_ref,
               x_ref, root_ref, rule_ref, op_ref, split_ref):
    x = lax.dot_general(predb_ref[...],
                        g_ref[...].astype(jnp.bfloat16),
                        (((1,), (1,)), ((), ())),
                        preferred_element_type=jnp.float32)
    x = x + bv_ref[...].reshape(1, B) - lse_ref[...]
    x_ref[...] = x.reshape(1, QALL, B)

    @pl.when(pl.program_id(0) == 0)
    def _():
        root_ref[...] = _log_softmax_rows(root_W_ref[...] + root_b_ref[...])
        rule_ref[...] = _log_softmax_rows(rule_W_ref[...] + rule_b_ref[...])
        op_ref[...] = _log_softmax_rows(op_W_ref[...] + op_b_ref[...])
        pred = pred_ref[...]
        h = jnp.dot(pred, s_in_W_ref[...],
                    preferred_element_type=jnp.float32) + s_in_b_ref[...]
        t = jax.nn.relu(jnp.dot(h, r1_W1_ref[...],
                                preferred_element_type=jnp.float32)
                        + r1_b1_ref[...])
        h = h + jax.nn.relu(jnp.dot(t, r1_W2_ref[...],
                                    preferred_element_type=jnp.float32)
                            + r1_b2_ref[...])
        t = jax.nn.relu(jnp.dot(h, r2_W1_ref[...],
                                preferred_element_type=jnp.float32)
                        + r2_b1_ref[...])
        h = h + jax.nn.relu(jnp.dot(t, r2_W2_ref[...],
                                    preferred_element_type=jnp.float32)
                            + r2_b2_ref[...])
        sp = jnp.dot(h, s_out_W_ref[...],
                     preferred_element_type=jnp.float32) + s_out_b_ref[...]
        split_ref[...] = _log_softmax_rows(sp)


def _emit(g, bv2, predb, predcat_emb, lse, root_W, root_b2, rule_W, rule_b2,
          op_W, op_b2, s_in_W, s_in_b2, r1_W1, r1_b12, r1_W2, r1_b22,
          r2_W1, r2_b12, r2_W2, r2_b22, s_out_W, s_out_b2):
    full = lambda shape: pl.BlockSpec(shape, lambda i: (0,) * len(shape))
    return pl.pallas_call(
        _emit_body,
        grid=(L,),
        in_specs=[
            pl.BlockSpec((B, STATE), lambda i: (i, 0)),
            pl.BlockSpec((1, 1, B), lambda i: (i, 0, 0)),
            full((QALL, STATE)),
            full((QALL, STATE)),
            full((QALL, 1)),
            full((1, QALL)), full((1, QALL)),
            full(rule_W.shape), full((1, rule_W.shape[1])),
            full(op_W.shape), full((1, op_W.shape[1])),
            full(s_in_W.shape), full((1, STATE)),
            full(r1_W1.shape), full((1, STATE)),
            full(r1_W2.shape), full((1, STATE)),
            full(r2_W1.shape), full((1, STATE)),
            full(r2_W2.shape), full((1, STATE)),
            full(s_out_W.shape), full((1, s_out_W.shape[1])),
        ],
        out_specs=[
            pl.BlockSpec((1, QALL, B), lambda i: (i, 0, 0)),
            full((1, QALL)),
            full(rule_W.shape),
            full(op_W.shape),
            full((QALL, s_out_W.shape[1])),
        ],
        out_shape=[
            jax.ShapeDtypeStruct((L, QALL, B), jnp.float32),
            jax.ShapeDtypeStruct((1, QALL), jnp.float32),
            jax.ShapeDtypeStruct(rule_W.shape, jnp.float32),
            jax.ShapeDtypeStruct(op_W.shape, jnp.float32),
            jax.ShapeDtypeStruct((QALL, s_out_W.shape[1]), jnp.float32),
        ],
    )(g, bv2, predb, predcat_emb, lse, root_W, root_b2, rule_W, rule_b2,
      op_W, op_b2, s_in_W, s_in_b2, r1_W1, r1_b12, r1_W2, r1_b22,
      r2_W1, r2_b12, r2_W2, r2_b22, s_out_W, s_out_b2)


def kernel(words, emit_W, emit_b, predcat_emb, root_W, root_b, rule_W, rule_b,
           op_W, op_b, s_in_W, s_in_b, r1_W1, r1_b1, r1_W2, r1_b2,
           r2_W1, r2_b1, r2_W2, r2_b2, s_out_W, s_out_b):
    emit_wt = emit_W.T
    vpad = NCHUNK * VC
    z2 = jnp.pad(jnp.exp(emit_b), (0, vpad - VOCAB)).reshape(vpad, 1)
    lse = _lse(predcat_emb, emit_wt, z2)
    words_flat = words.T.reshape(NWORDS).astype(jnp.int32)
    g, bv = _sc_gather(emit_wt, emit_b, words_flat)
    x_p, root, rule, op, split = _emit(
        g, bv.reshape(L, 1, B), predcat_emb.astype(jnp.bfloat16),
        predcat_emb, lse,
        root_W, root_b.reshape(1, QALL), rule_W, rule_b.reshape(1, -1),
        op_W, op_b.reshape(1, -1), s_in_W, s_in_b.reshape(1, -1),
        r1_W1, r1_b1.reshape(1, -1), r1_W2, r1_b2.reshape(1, -1),
        r2_W1, r2_b1.reshape(1, -1), r2_W2, r2_b2.reshape(1, -1),
        s_out_W, s_out_b.reshape(1, -1))
    x = jnp.transpose(x_p, (2, 0, 1))
    return (x, root.reshape(QALL), rule, op, split)


# speedup vs baseline: 1.2288x; 1.2288x over previous
import functools

import jax
import jax.numpy as jnp
from jax import lax
from jax.experimental import pallas as pl
from jax.experimental.pallas import tpu as pltpu
from jax.experimental.pallas import tpu_sc as plsc

STATE = 128
VOCAB = 100000
QALL = 300
B = 1024
L = 50
NWORDS = B * L

VC = 4096
NCHUNK = -(-VOCAB // VC)

_NEG = -1e30


def _lse_body(pred_ref, wt_ref, b_ref, lse_ref, m_ref, s_ref):
    i = pl.program_id(0)
    logits = lax.dot_general(pred_ref[...], wt_ref[...],
                             (((1,), (1,)), ((), ())),
                             preferred_element_type=jnp.float32)
    logits = logits + b_ref[...]

    @pl.when(i == 0)
    def _():
        m_ref[...] = jnp.full((QALL, 1), _NEG, jnp.float32)
        s_ref[...] = jnp.zeros((QALL, 1), jnp.float32)

    def update(lm):
        m_old = m_ref[...]
        s_old = s_ref[...]
        m_new = jnp.maximum(m_old, jnp.max(lm, axis=1, keepdims=True))
        s_new = s_old * jnp.exp(m_old - m_new) + jnp.sum(
            jnp.exp(lm - m_new), axis=1, keepdims=True)
        m_ref[...] = m_new
        s_ref[...] = s_new
        return m_new, s_new

    @pl.when(i < NCHUNK - 1)
    def _():
        update(logits)

    @pl.when(i == NCHUNK - 1)
    def _():
        col = i * VC + lax.broadcasted_iota(jnp.int32, (1, VC), 1)
        m_new, s_new = update(jnp.where(col < VOCAB, logits, _NEG))
        lse_ref[...] = m_new + jnp.log(s_new)


def _lse(predcat_emb, emit_wt, emit_b2):
    return pl.pallas_call(
        _lse_body,
        grid=(NCHUNK,),
        in_specs=[
            pl.BlockSpec((QALL, STATE), lambda i: (0, 0)),
            pl.BlockSpec((VC, STATE), lambda i: (i, 0)),
            pl.BlockSpec((1, VC), lambda i: (0, i)),
        ],
        out_specs=pl.BlockSpec((QALL, 1), lambda i: (0, 0)),
        out_shape=jax.ShapeDtypeStruct((QALL, 1), jnp.float32),
        scratch_shapes=[
            pltpu.VMEM((QALL, 1), jnp.float32),
            pltpu.VMEM((QALL, 1), jnp.float32),
        ],
    )(predcat_emb, emit_wt, emit_b2)


def _sc_gather(wt, emit_b, words_flat):
    info = plsc.get_sparse_core_info()
    nc, ns = info.num_cores, info.num_subcores
    nw = nc * ns
    b_per_w = NWORDS // nw
    nchunk = 4
    ch = b_per_w // nchunk

    mesh = plsc.VectorSubcoreMesh(core_axis_name="c", subcore_axis_name="s")

    @functools.partial(
        pl.kernel, mesh=mesh,
        out_type=[
            jax.ShapeDtypeStruct((NWORDS, STATE), jnp.float32),
            jax.ShapeDtypeStruct((NWORDS,), jnp.float32),
        ],
        scratch_types=[
            pltpu.VMEM((b_per_w,), jnp.int32),
            pltpu.VMEM((ch, STATE), jnp.float32),
            pltpu.VMEM((ch, STATE), jnp.float32),
            pltpu.VMEM((ch,), jnp.float32),
            pltpu.VMEM((ch,), jnp.float32),
            pltpu.SemaphoreType.DMA,
            pltpu.SemaphoreType.DMA,
        ],
    )
    def k(wt_hbm, b_hbm, words_hbm, g_hbm, bv_hbm, idx_v, rows_v0, rows_v1,
          brow_v0, brow_v1, sem_r, sem_b):
        wid = lax.axis_index("s") * nc + lax.axis_index("c")
        base = wid * b_per_w
        rows_v = (rows_v0, rows_v1)
        brow_v = (brow_v0, brow_v1)
        pltpu.sync_copy(words_hbm.at[pl.ds(base, b_per_w)], idx_v)

        def fire(c, slot):
            idx_c = idx_v.at[pl.ds(c * ch, ch)]
            pltpu.async_copy(wt_hbm.at[idx_c], rows_v[slot], sem_r)
            pltpu.async_copy(b_hbm.at[idx_c], brow_v[slot], sem_b)

        def drain(c, slot):
            pltpu.make_async_copy(
                wt_hbm.at[idx_v.at[pl.ds(c * ch, ch)]], rows_v[slot],
                sem_r).wait()
            pltpu.make_async_copy(
                b_hbm.at[idx_v.at[pl.ds(c * ch, ch)]], brow_v[slot],
                sem_b).wait()
            pltpu.sync_copy(rows_v[slot],
                            g_hbm.at[pl.ds(base + c * ch, ch)])
            pltpu.sync_copy(brow_v[slot],
                            bv_hbm.at[pl.ds(base + c * ch, ch)])

        fire(0, 0)
        for c in range(nchunk):
            if c + 1 < nchunk:
                fire(c + 1, (c + 1) % 2)
            drain(c, c % 2)

    return k(wt, emit_b, words_flat)


def _log_softmax_rows(x):
    m = jnp.max(x, axis=1, keepdims=True)
    return x - m - jnp.log(jnp.sum(jnp.exp(x - m), axis=1, keepdims=True))


def _emit_body(g_ref, bv_ref, predb_ref, pred_ref, lse_ref,
               root_W_ref, root_b_ref, rule_W_ref, rule_b_ref,
               op_W_ref, op_b_ref, s_in_W_ref, s_in_b_ref,
               r1_W1_ref, r1_b1_ref, r1_W2_ref, r1_b2_ref,
               r2_W1_ref, r2_b1_ref, r2_W2_ref, r2_b2_ref,
               s_out_W_ref, s_out_b_ref,
               x_ref, root_ref, rule_ref, op_ref, split_ref):
    x = lax.dot_general(predb_ref[...],
                        g_ref[...].astype(jnp.bfloat16),
                        (((1,), (1,)), ((), ())),
                        preferred_element_type=jnp.float32)
    x = x + bv_ref[...].reshape(1, B) - lse_ref[...]
    x_ref[...] = x.reshape(1, QALL, B)

    @pl.when(pl.program_id(0) == 0)
    def _():
        root_ref[...] = _log_softmax_rows(root_W_ref[...] + root_b_ref[...])
        rule_ref[...] = _log_softmax_rows(rule_W_ref[...] + rule_b_ref[...])
        op_ref[...] = _log_softmax_rows(op_W_ref[...] + op_b_ref[...])
        pred = pred_ref[...]
        h = jnp.dot(pred, s_in_W_ref[...],
                    preferred_element_type=jnp.float32) + s_in_b_ref[...]
        t = jax.nn.relu(jnp.dot(h, r1_W1_ref[...],
                                preferred_element_type=jnp.float32)
                        + r1_b1_ref[...])
        h = h + jax.nn.relu(jnp.dot(t, r1_W2_ref[...],
                                    preferred_element_type=jnp.float32)
                            + r1_b2_ref[...])
        t = jax.nn.relu(jnp.dot(h, r2_W1_ref[...],
                                preferred_element_type=jnp.float32)
                        + r2_b1_ref[...])
        h = h + jax.nn.relu(jnp.dot(t, r2_W2_ref[...],
                                    preferred_element_type=jnp.float32)
                            + r2_b2_ref[...])
        sp = jnp.dot(h, s_out_W_ref[...],
                     preferred_element_type=jnp.float32) + s_out_b_ref[...]
        split_ref[...] = _log_softmax_rows(sp)


def _emit(g, bv2, predb, predcat_emb, lse, root_W, root_b2, rule_W, rule_b2,
          op_W, op_b2, s_in_W, s_in_b2, r1_W1, r1_b12, r1_W2, r1_b22,
          r2_W1, r2_b12, r2_W2, r2_b22, s_out_W, s_out_b2):
    full = lambda shape: pl.BlockSpec(shape, lambda i: (0,) * len(shape))
    return pl.pallas_call(
        _emit_body,
        grid=(L,),
        in_specs=[
            pl.BlockSpec((B, STATE), lambda i: (i, 0)),
            pl.BlockSpec((1, 1, B), lambda i: (i, 0, 0)),
            full((QALL, STATE)),
            full((QALL, STATE)),
            full((QALL, 1)),
            full((1, QALL)), full((1, QALL)),
            full(rule_W.shape), full((1, rule_W.shape[1])),
            full(op_W.shape), full((1, op_W.shape[1])),
            full(s_in_W.shape), full((1, STATE)),
            full(r1_W1.shape), full((1, STATE)),
            full(r1_W2.shape), full((1, STATE)),
            full(r2_W1.shape), full((1, STATE)),
            full(r2_W2.shape), full((1, STATE)),
            full(s_out_W.shape), full((1, s_out_W.shape[1])),
        ],
        out_specs=[
            pl.BlockSpec((1, QALL, B), lambda i: (i, 0, 0)),
            full((1, QALL)),
            full(rule_W.shape),
            full(op_W.shape),
            full((QALL, s_out_W.shape[1])),
        ],
        out_shape=[
            jax.ShapeDtypeStruct((L, QALL, B), jnp.float32),
            jax.ShapeDtypeStruct((1, QALL), jnp.float32),
            jax.ShapeDtypeStruct(rule_W.shape, jnp.float32),
            jax.ShapeDtypeStruct(op_W.shape, jnp.float32),
            jax.ShapeDtypeStruct((QALL, s_out_W.shape[1]), jnp.float32),
        ],
    )(g, bv2, predb, predcat_emb, lse, root_W, root_b2, rule_W, rule_b2,
      op_W, op_b2, s_in_W, s_in_b2, r1_W1, r1_b12, r1_W2, r1_b22,
      r2_W1, r2_b12, r2_W2, r2_b22, s_out_W, s_out_b2)


def kernel(words, emit_W, emit_b, predcat_emb, root_W, root_b, rule_W, rule_b,
           op_W, op_b, s_in_W, s_in_b, r1_W1, r1_b1, r1_W2, r1_b2,
           r2_W1, r2_b1, r2_W2, r2_b2, s_out_W, s_out_b):
    emit_wt = emit_W.T
    lse = _lse(predcat_emb, emit_wt, emit_b.reshape(1, VOCAB))
    words_flat = words.T.reshape(NWORDS).astype(jnp.int32)
    g, bv = _sc_gather(emit_wt, emit_b, words_flat)
    x_p, root, rule, op, split = _emit(
        g, bv.reshape(L, 1, B), predcat_emb.astype(jnp.bfloat16),
        predcat_emb, lse,
        root_W, root_b.reshape(1, QALL), rule_W, rule_b.reshape(1, -1),
        op_W, op_b.reshape(1, -1), s_in_W, s_in_b.reshape(1, -1),
        r1_W1, r1_b1.reshape(1, -1), r1_W2, r1_b2.reshape(1, -1),
        r2_W1, r2_b1.reshape(1, -1), r2_W2, r2_b2.reshape(1, -1),
        s_out_W, s_out_b.reshape(1, -1))
    x = jnp.transpose(x_p, (2, 0, 1))
    return (x, root.reshape(QALL), rule, op, split)


# speedup vs baseline: 1.4876x; 1.2107x over previous
import functools

import jax
import jax.numpy as jnp
from jax import lax
from jax.experimental import pallas as pl
from jax.experimental.pallas import tpu as pltpu
from jax.experimental.pallas import tpu_sc as plsc

STATE = 128
VOCAB = 100000
QALL = 300
B = 1024
L = 50
NWORDS = B * L

VC = 8192
NCHUNK = -(-VOCAB // VC)
LB = 5
NLB = L // LB

_NEG = -1e30


def _lse_body(pred_ref, wt_ref, b_ref, lse_ref, m_ref, s_ref):
    i = pl.program_id(0)
    logits = lax.dot_general(pred_ref[...], wt_ref[...],
                             (((1,), (1,)), ((), ())),
                             preferred_element_type=jnp.float32)
    logits = logits + b_ref[...]

    @pl.when(i == 0)
    def _():
        m_ref[...] = jnp.full((QALL, 1), _NEG, jnp.float32)
        s_ref[...] = jnp.zeros((QALL, 1), jnp.float32)

    def update(lm):
        m_old = m_ref[...]
        s_old = s_ref[...]
        m_new = jnp.maximum(m_old, jnp.max(lm, axis=1, keepdims=True))
        s_new = s_old * jnp.exp(m_old - m_new) + jnp.sum(
            jnp.exp(lm - m_new), axis=1, keepdims=True)
        m_ref[...] = m_new
        s_ref[...] = s_new
        return m_new, s_new

    @pl.when(i < NCHUNK - 1)
    def _():
        update(logits)

    @pl.when(i == NCHUNK - 1)
    def _():
        col = i * VC + lax.broadcasted_iota(jnp.int32, (1, VC), 1)
        m_new, s_new = update(jnp.where(col < VOCAB, logits, _NEG))
        lse_ref[...] = m_new + jnp.log(s_new)


def _lse(predcat_emb, emit_wt, emit_b2):
    return pl.pallas_call(
        _lse_body,
        grid=(NCHUNK,),
        in_specs=[
            pl.BlockSpec((QALL, STATE), lambda i: (0, 0)),
            pl.BlockSpec((VC, STATE), lambda i: (i, 0)),
            pl.BlockSpec((1, VC), lambda i: (0, i)),
        ],
        out_specs=pl.BlockSpec((QALL, 1), lambda i: (0, 0)),
        out_shape=jax.ShapeDtypeStruct((QALL, 1), jnp.float32),
        scratch_shapes=[
            pltpu.VMEM((QALL, 1), jnp.float32),
            pltpu.VMEM((QALL, 1), jnp.float32),
        ],
    )(predcat_emb, emit_wt, emit_b2)


def _sc_gather(wt, emit_b, words_flat):
    info = plsc.get_sparse_core_info()
    nc, ns = info.num_cores, info.num_subcores
    nw = nc * ns
    b_per_w = NWORDS // nw
    nchunk = 4
    ch = b_per_w // nchunk

    mesh = plsc.VectorSubcoreMesh(core_axis_name="c", subcore_axis_name="s")

    @functools.partial(
        pl.kernel, mesh=mesh,
        out_type=[
            jax.ShapeDtypeStruct((NWORDS, STATE), jnp.float32),
            jax.ShapeDtypeStruct((NWORDS,), jnp.float32),
        ],
        scratch_types=[
            pltpu.VMEM((b_per_w,), jnp.int32),
            pltpu.VMEM((ch, STATE), jnp.float32),
            pltpu.VMEM((ch, STATE), jnp.float32),
            pltpu.VMEM((ch,), jnp.float32),
            pltpu.VMEM((ch,), jnp.float32),
            pltpu.SemaphoreType.DMA,
            pltpu.SemaphoreType.DMA,
        ],
    )
    def k(wt_hbm, b_hbm, words_hbm, g_hbm, bv_hbm, idx_v, rows_v0, rows_v1,
          brow_v0, brow_v1, sem_r, sem_b):
        wid = lax.axis_index("s") * nc + lax.axis_index("c")
        base = wid * b_per_w
        rows_v = (rows_v0, rows_v1)
        brow_v = (brow_v0, brow_v1)
        pltpu.sync_copy(words_hbm.at[pl.ds(base, b_per_w)], idx_v)

        def fire(c, slot):
            idx_c = idx_v.at[pl.ds(c * ch, ch)]
            pltpu.async_copy(wt_hbm.at[idx_c], rows_v[slot], sem_r)
            pltpu.async_copy(b_hbm.at[idx_c], brow_v[slot], sem_b)

        def drain(c, slot):
            pltpu.make_async_copy(
                wt_hbm.at[idx_v.at[pl.ds(c * ch, ch)]], rows_v[slot],
                sem_r).wait()
            pltpu.make_async_copy(
                b_hbm.at[idx_v.at[pl.ds(c * ch, ch)]], brow_v[slot],
                sem_b).wait()
            pltpu.sync_copy(rows_v[slot],
                            g_hbm.at[pl.ds(base + c * ch, ch)])
            pltpu.sync_copy(brow_v[slot],
                            bv_hbm.at[pl.ds(base + c * ch, ch)])

        fire(0, 0)
        for c in range(nchunk):
            if c + 1 < nchunk:
                fire(c + 1, (c + 1) % 2)
            drain(c, c % 2)

    return k(wt, emit_b, words_flat)


def _log_softmax_rows(x):
    m = jnp.max(x, axis=1, keepdims=True)
    return x - m - jnp.log(jnp.sum(jnp.exp(x - m), axis=1, keepdims=True))


def _emit_body(g_ref, bv_ref, predb_ref, pred_ref, lse_ref,
               root_W_ref, root_b_ref, rule_W_ref, rule_b_ref,
               op_W_ref, op_b_ref, s_in_W_ref, s_in_b_ref,
               r1_W1_ref, r1_b1_ref, r1_W2_ref, r1_b2_ref,
               r2_W1_ref, r2_b1_ref, r2_W2_ref, r2_b2_ref,
               s_out_W_ref, s_out_b_ref,
               x_ref, root_ref, rule_ref, op_ref, split_ref):
    predb = predb_ref[...]
    lse = lse_ref[...]
    for p in range(LB):
        x = lax.dot_general(predb,
                            g_ref[p * B:(p + 1) * B, :].astype(jnp.bfloat16),
                            (((1,), (1,)), ((), ())),
                            preferred_element_type=jnp.float32)
        x = x + bv_ref[p].reshape(1, B) - lse
        x_ref[p] = x

    @pl.when(pl.program_id(0) == 0)
    def _():
        root_ref[...] = _log_softmax_rows(root_W_ref[...] + root_b_ref[...])
        rule_ref[...] = _log_softmax_rows(rule_W_ref[...] + rule_b_ref[...])
        op_ref[...] = _log_softmax_rows(op_W_ref[...] + op_b_ref[...])
        pred = pred_ref[...]
        h = jnp.dot(pred, s_in_W_ref[...],
                    preferred_element_type=jnp.float32) + s_in_b_ref[...]
        t = jax.nn.relu(jnp.dot(h, r1_W1_ref[...],
                                preferred_element_type=jnp.float32)
                        + r1_b1_ref[...])
        h = h + jax.nn.relu(jnp.dot(t, r1_W2_ref[...],
                                    preferred_element_type=jnp.float32)
                            + r1_b2_ref[...])
        t = jax.nn.relu(jnp.dot(h, r2_W1_ref[...],
                                preferred_element_type=jnp.float32)
                        + r2_b1_ref[...])
        h = h + jax.nn.relu(jnp.dot(t, r2_W2_ref[...],
                                    preferred_element_type=jnp.float32)
                            + r2_b2_ref[...])
        sp = jnp.dot(h, s_out_W_ref[...],
                     preferred_element_type=jnp.float32) + s_out_b_ref[...]
        split_ref[...] = _log_softmax_rows(sp)


def _emit(g, bv2, predb, predcat_emb, lse, root_W, root_b2, rule_W, rule_b2,
          op_W, op_b2, s_in_W, s_in_b2, r1_W1, r1_b12, r1_W2, r1_b22,
          r2_W1, r2_b12, r2_W2, r2_b22, s_out_W, s_out_b2):
    full = lambda shape: pl.BlockSpec(shape, lambda i: (0,) * len(shape))
    return pl.pallas_call(
        _emit_body,
        grid=(NLB,),
        in_specs=[
            pl.BlockSpec((LB * B, STATE), lambda i: (i, 0)),
            pl.BlockSpec((LB, 1, B), lambda i: (i, 0, 0)),
            full((QALL, STATE)),
            full((QALL, STATE)),
            full((QALL, 1)),
            full((1, QALL)), full((1, QALL)),
            full(rule_W.shape), full((1, rule_W.shape[1])),
            full(op_W.shape), full((1, op_W.shape[1])),
            full(s_in_W.shape), full((1, STATE)),
            full(r1_W1.shape), full((1, STATE)),
            full(r1_W2.shape), full((1, STATE)),
            full(r2_W1.shape), full((1, STATE)),
            full(r2_W2.shape), full((1, STATE)),
            full(s_out_W.shape), full((1, s_out_W.shape[1])),
        ],
        out_specs=[
            pl.BlockSpec((LB, QALL, B), lambda i: (i, 0, 0)),
            full((1, QALL)),
            full(rule_W.shape),
            full(op_W.shape),
            full((QALL, s_out_W.shape[1])),
        ],
        out_shape=[
            jax.ShapeDtypeStruct((L, QALL, B), jnp.float32),
            jax.ShapeDtypeStruct((1, QALL), jnp.float32),
            jax.ShapeDtypeStruct(rule_W.shape, jnp.float32),
            jax.ShapeDtypeStruct(op_W.shape, jnp.float32),
            jax.ShapeDtypeStruct((QALL, s_out_W.shape[1]), jnp.float32),
        ],
    )(g, bv2, predb, predcat_emb, lse, root_W, root_b2, rule_W, rule_b2,
      op_W, op_b2, s_in_W, s_in_b2, r1_W1, r1_b12, r1_W2, r1_b22,
      r2_W1, r2_b12, r2_W2, r2_b22, s_out_W, s_out_b2)


def kernel(words, emit_W, emit_b, predcat_emb, root_W, root_b, rule_W, rule_b,
           op_W, op_b, s_in_W, s_in_b, r1_W1, r1_b1, r1_W2, r1_b2,
           r2_W1, r2_b1, r2_W2, r2_b2, s_out_W, s_out_b):
    emit_wt = emit_W.T
    lse = _lse(predcat_emb, emit_wt, emit_b.reshape(1, VOCAB))
    words_flat = words.T.reshape(NWORDS).astype(jnp.int32)
    g, bv = _sc_gather(emit_wt, emit_b, words_flat)
    x_p, root, rule, op, split = _emit(
        g, bv.reshape(L, 1, B), predcat_emb.astype(jnp.bfloat16),
        predcat_emb, lse,
        root_W, root_b.reshape(1, QALL), rule_W, rule_b.reshape(1, -1),
        op_W, op_b.reshape(1, -1), s_in_W, s_in_b.reshape(1, -1),
        r1_W1, r1_b1.reshape(1, -1), r1_W2, r1_b2.reshape(1, -1),
        r2_W1, r2_b1.reshape(1, -1), r2_W2, r2_b2.reshape(1, -1),
        s_out_W, s_out_b.reshape(1, -1))
    x = jnp.transpose(x_p, (2, 0, 1))
    return (x, root.reshape(QALL), rule, op, split)


# speedup vs baseline: 1.5262x; 1.0259x over previous
import functools

import jax
import jax.numpy as jnp
from jax import lax
from jax.experimental import pallas as pl
from jax.experimental.pallas import tpu as pltpu
from jax.experimental.pallas import tpu_sc as plsc

STATE = 128
VOCAB = 100000
QALL = 300
B = 1024
L = 50
NWORDS = B * L

VC = 8192
NCHUNK = -(-VOCAB // VC)
LB = 5
NLB = L // LB

_NEG = -1e30


def _lse_body(pred_ref, wt_ref, b_ref, lse_ref, m_ref, s_ref):
    i = pl.program_id(0)
    logits = lax.dot_general(pred_ref[...], wt_ref[...],
                             (((1,), (1,)), ((), ())),
                             preferred_element_type=jnp.float32)
    logits = logits + b_ref[...]

    @pl.when(i == 0)
    def _():
        m_ref[...] = jnp.full((QALL, 1), _NEG, jnp.float32)
        s_ref[...] = jnp.zeros((QALL, 1), jnp.float32)

    def update(lm):
        m_old = m_ref[...]
        s_old = s_ref[...]
        m_new = jnp.maximum(m_old, jnp.max(lm, axis=1, keepdims=True))
        s_new = s_old * jnp.exp(m_old - m_new) + jnp.sum(
            jnp.exp(lm - m_new), axis=1, keepdims=True)
        m_ref[...] = m_new
        s_ref[...] = s_new
        return m_new, s_new

    @pl.when(i < NCHUNK - 1)
    def _():
        update(logits)

    @pl.when(i == NCHUNK - 1)
    def _():
        col = i * VC + lax.broadcasted_iota(jnp.int32, (1, VC), 1)
        m_new, s_new = update(jnp.where(col < VOCAB, logits, _NEG))
        lse_ref[...] = m_new + jnp.log(s_new)


def _lse(predcat_emb, emit_wt, emit_b2):
    return pl.pallas_call(
        _lse_body,
        grid=(NCHUNK,),
        in_specs=[
            pl.BlockSpec((QALL, STATE), lambda i: (0, 0)),
            pl.BlockSpec((VC, STATE), lambda i: (i, 0)),
            pl.BlockSpec((1, VC), lambda i: (0, i)),
        ],
        out_specs=pl.BlockSpec((QALL, 1), lambda i: (0, 0)),
        out_shape=jax.ShapeDtypeStruct((QALL, 1), jnp.float32),
        scratch_shapes=[
            pltpu.VMEM((QALL, 1), jnp.float32),
            pltpu.VMEM((QALL, 1), jnp.float32),
        ],
    )(predcat_emb, emit_wt, emit_b2)


def _sc_gather(wt, emit_b, words_flat):
    info = plsc.get_sparse_core_info()
    nc, ns = info.num_cores, info.num_subcores
    nw = nc * ns
    b_per_w = NWORDS // nw
    nchunk = 4
    ch = b_per_w // nchunk

    mesh = plsc.VectorSubcoreMesh(core_axis_name="c", subcore_axis_name="s")

    @functools.partial(
        pl.kernel, mesh=mesh,
        out_type=[
            jax.ShapeDtypeStruct((NWORDS, STATE), jnp.float32),
            jax.ShapeDtypeStruct((NWORDS,), jnp.float32),
        ],
        scratch_types=[
            pltpu.VMEM((b_per_w,), jnp.int32),
            pltpu.VMEM((ch, STATE), jnp.float32),
            pltpu.VMEM((ch, STATE), jnp.float32),
            pltpu.VMEM((ch,), jnp.float32),
            pltpu.VMEM((ch,), jnp.float32),
            pltpu.SemaphoreType.DMA,
            pltpu.SemaphoreType.DMA,
        ],
    )
    def k(wt_hbm, b_hbm, words_hbm, g_hbm, bv_hbm, idx_v, rows_v0, rows_v1,
          brow_v0, brow_v1, sem_r, sem_b):
        wid = lax.axis_index("s") * nc + lax.axis_index("c")
        base = wid * b_per_w
        rows_v = (rows_v0, rows_v1)
        brow_v = (brow_v0, brow_v1)
        pltpu.sync_copy(words_hbm.at[pl.ds(base, b_per_w)], idx_v)

        def fire(c, slot):
            idx_c = idx_v.at[pl.ds(c * ch, ch)]
            pltpu.async_copy(wt_hbm.at[idx_c], rows_v[slot], sem_r)
            pltpu.async_copy(b_hbm.at[idx_c], brow_v[slot], sem_b)

        def drain(c, slot):
            pltpu.make_async_copy(
                wt_hbm.at[idx_v.at[pl.ds(c * ch, ch)]], rows_v[slot],
                sem_r).wait()
            pltpu.make_async_copy(
                b_hbm.at[idx_v.at[pl.ds(c * ch, ch)]], brow_v[slot],
                sem_b).wait()
            pltpu.sync_copy(rows_v[slot],
                            g_hbm.at[pl.ds(base + c * ch, ch)])
            pltpu.sync_copy(brow_v[slot],
                            bv_hbm.at[pl.ds(base + c * ch, ch)])

        fire(0, 0)
        for c in range(nchunk):
            if c + 1 < nchunk:
                fire(c + 1, (c + 1) % 2)
            drain(c, c % 2)

    return k(wt, emit_b, words_flat)


def _log_softmax_rows(x):
    m = jnp.max(x, axis=1, keepdims=True)
    return x - m - jnp.log(jnp.sum(jnp.exp(x - m), axis=1, keepdims=True))


def _emit_body(g_ref, bv_ref, predb_ref, pred_ref, lse_ref,
               root_W_ref, root_b_ref, rule_W_ref, rule_b_ref,
               op_W_ref, op_b_ref, s_in_W_ref, s_in_b_ref,
               r1_W1_ref, r1_b1_ref, r1_W2_ref, r1_b2_ref,
               r2_W1_ref, r2_b1_ref, r2_W2_ref, r2_b2_ref,
               s_out_W_ref, s_out_b_ref,
               x_ref, root_ref, rule_ref, op_ref, split_ref):
    predb = predb_ref[...]
    lse = lse_ref[...]
    for p in range(LB):
        x = lax.dot_general(predb,
                            g_ref[p * B:(p + 1) * B, :].astype(jnp.bfloat16),
                            (((1,), (1,)), ((), ())),
                            preferred_element_type=jnp.float32)
        x = x + bv_ref[p].reshape(1, B) - lse
        x_ref[p] = x

    @pl.when(pl.program_id(0) == 0)
    def _():
        root_ref[...] = _log_softmax_rows(root_W_ref[...] + root_b_ref[...])
        rule_ref[...] = _log_softmax_rows(rule_W_ref[...] + rule_b_ref[...])
        op_ref[...] = _log_softmax_rows(op_W_ref[...] + op_b_ref[...]).T
        pred = pred_ref[...]
        h = jnp.dot(pred, s_in_W_ref[...],
                    preferred_element_type=jnp.float32) + s_in_b_ref[...]
        t = jax.nn.relu(jnp.dot(h, r1_W1_ref[...],
                                preferred_element_type=jnp.float32)
                        + r1_b1_ref[...])
        h = h + jax.nn.relu(jnp.dot(t, r1_W2_ref[...],
                                    preferred_element_type=jnp.float32)
                            + r1_b2_ref[...])
        t = jax.nn.relu(jnp.dot(h, r2_W1_ref[...],
                                preferred_element_type=jnp.float32)
                        + r2_b1_ref[...])
        h = h + jax.nn.relu(jnp.dot(t, r2_W2_ref[...],
                                    preferred_element_type=jnp.float32)
                            + r2_b2_ref[...])
        sp = jnp.dot(h, s_out_W_ref[...],
                     preferred_element_type=jnp.float32) + s_out_b_ref[...]
        split_ref[...] = _log_softmax_rows(sp).T


def _emit(g, bv2, predb, predcat_emb, lse, root_W, root_b2, rule_W, rule_b2,
          op_W, op_b2, s_in_W, s_in_b2, r1_W1, r1_b12, r1_W2, r1_b22,
          r2_W1, r2_b12, r2_W2, r2_b22, s_out_W, s_out_b2):
    full = lambda shape: pl.BlockSpec(shape, lambda i: (0,) * len(shape))
    return pl.pallas_call(
        _emit_body,
        grid=(NLB,),
        in_specs=[
            pl.BlockSpec((LB * B, STATE), lambda i: (i, 0)),
            pl.BlockSpec((LB, 1, B), lambda i: (i, 0, 0)),
            full((QALL, STATE)),
            full((QALL, STATE)),
            full((QALL, 1)),
            full((1, QALL)), full((1, QALL)),
            full(rule_W.shape), full((1, rule_W.shape[1])),
            full(op_W.shape), full((1, op_W.shape[1])),
            full(s_in_W.shape), full((1, STATE)),
            full(r1_W1.shape), full((1, STATE)),
            full(r1_W2.shape), full((1, STATE)),
            full(r2_W1.shape), full((1, STATE)),
            full(r2_W2.shape), full((1, STATE)),
            full(s_out_W.shape), full((1, s_out_W.shape[1])),
        ],
        out_specs=[
            pl.BlockSpec((LB, QALL, B), lambda i: (i, 0, 0)),
            full((1, QALL)),
            full(rule_W.shape),
            full(op_W.shape[::-1]),
            full((s_out_W.shape[1], QALL)),
        ],
        out_shape=[
            jax.ShapeDtypeStruct((L, QALL, B), jnp.float32),
            jax.ShapeDtypeStruct((1, QALL), jnp.float32),
            jax.ShapeDtypeStruct(rule_W.shape, jnp.float32),
            jax.ShapeDtypeStruct(op_W.shape[::-1], jnp.float32),
            jax.ShapeDtypeStruct((s_out_W.shape[1], QALL), jnp.float32),
        ],
    )(g, bv2, predb, predcat_emb, lse, root_W, root_b2, rule_W, rule_b2,
      op_W, op_b2, s_in_W, s_in_b2, r1_W1, r1_b12, r1_W2, r1_b22,
      r2_W1, r2_b12, r2_W2, r2_b22, s_out_W, s_out_b2)


def kernel(words, emit_W, emit_b, predcat_emb, root_W, root_b, rule_W, rule_b,
           op_W, op_b, s_in_W, s_in_b, r1_W1, r1_b1, r1_W2, r1_b2,
           r2_W1, r2_b1, r2_W2, r2_b2, s_out_W, s_out_b):
    emit_wt = emit_W.T
    lse = _lse(predcat_emb, emit_wt, emit_b.reshape(1, VOCAB))
    words_flat = words.T.reshape(NWORDS).astype(jnp.int32)
    g, bv = _sc_gather(emit_wt, emit_b, words_flat)
    x_p, root, rule, op, split = _emit(
        g, bv.reshape(L, 1, B), predcat_emb.astype(jnp.bfloat16),
        predcat_emb, lse,
        root_W, root_b.reshape(1, QALL), rule_W, rule_b.reshape(1, -1),
        op_W, op_b.reshape(1, -1), s_in_W, s_in_b.reshape(1, -1),
        r1_W1, r1_b1.reshape(1, -1), r1_W2, r1_b2.reshape(1, -1),
        r2_W1, r2_b1.reshape(1, -1), r2_W2, r2_b2.reshape(1, -1),
        s_out_W, s_out_b.reshape(1, -1))
    x = jnp.transpose(x_p, (2, 0, 1))
    return (x, root.reshape(QALL), rule, op.T, split.T)


# speedup vs baseline: 1.5344x; 1.0053x over previous
import functools

import jax
import jax.numpy as jnp
from jax import lax
from jax.experimental import pallas as pl
from jax.experimental.pallas import tpu as pltpu
from jax.experimental.pallas import tpu_sc as plsc

STATE = 128
VOCAB = 100000
QALL = 300
B = 1024
L = 50
NWORDS = B * L

VC = 10000
NCHUNK = VOCAB // VC
LB = 5
NLB = L // LB

_NEG = -1e30


def _lse_body(pred_ref, wt_ref, b_ref, lse_ref, m_ref, s_ref):
    i = pl.program_id(0)
    logits = lax.dot_general(pred_ref[...], wt_ref[...],
                             (((1,), (1,)), ((), ())),
                             preferred_element_type=jnp.float32)
    logits = logits + b_ref[...].reshape(1, VC)

    @pl.when(i == 0)
    def _():
        m_ref[...] = jnp.full((QALL, 1), _NEG, jnp.float32)
        s_ref[...] = jnp.zeros((QALL, 1), jnp.float32)

    def update(lm):
        m_old = m_ref[...]
        s_old = s_ref[...]
        m_new = jnp.maximum(m_old, jnp.max(lm, axis=1, keepdims=True))
        s_new = s_old * jnp.exp(m_old - m_new) + jnp.sum(
            jnp.exp(lm - m_new), axis=1, keepdims=True)
        m_ref[...] = m_new
        s_ref[...] = s_new
        return m_new, s_new

    m_new, s_new = update(logits)

    @pl.when(i == NCHUNK - 1)
    def _():
        lse_ref[...] = m_new + jnp.log(s_new)


def _lse(predcat_emb, emit_wt, emit_b2):
    return pl.pallas_call(
        _lse_body,
        grid=(NCHUNK,),
        in_specs=[
            pl.BlockSpec((QALL, STATE), lambda i: (0, 0)),
            pl.BlockSpec((VC, STATE), lambda i: (i, 0)),
            pl.BlockSpec((1, 1, VC), lambda i: (i, 0, 0)),
        ],
        out_specs=pl.BlockSpec((QALL, 1), lambda i: (0, 0)),
        out_shape=jax.ShapeDtypeStruct((QALL, 1), jnp.float32),
        scratch_shapes=[
            pltpu.VMEM((QALL, 1), jnp.float32),
            pltpu.VMEM((QALL, 1), jnp.float32),
        ],
    )(predcat_emb, emit_wt, emit_b2)


def _sc_gather(wt, emit_b, words_flat):
    info = plsc.get_sparse_core_info()
    nc, ns = info.num_cores, info.num_subcores
    nw = nc * ns
    b_per_w = NWORDS // nw
    nchunk = 4
    ch = b_per_w // nchunk

    mesh = plsc.VectorSubcoreMesh(core_axis_name="c", subcore_axis_name="s")

    @functools.partial(
        pl.kernel, mesh=mesh,
        out_type=[
            jax.ShapeDtypeStruct((NWORDS, STATE), jnp.float32),
            jax.ShapeDtypeStruct((NWORDS,), jnp.float32),
        ],
        scratch_types=[
            pltpu.VMEM((b_per_w,), jnp.int32),
            pltpu.VMEM((ch, STATE), jnp.float32),
            pltpu.VMEM((ch, STATE), jnp.float32),
            pltpu.VMEM((ch,), jnp.float32),
            pltpu.VMEM((ch,), jnp.float32),
            pltpu.SemaphoreType.DMA,
            pltpu.SemaphoreType.DMA,
        ],
    )
    def k(wt_hbm, b_hbm, words_hbm, g_hbm, bv_hbm, idx_v, rows_v0, rows_v1,
          brow_v0, brow_v1, sem_r, sem_b):
        wid = lax.axis_index("s") * nc + lax.axis_index("c")
        base = wid * b_per_w
        rows_v = (rows_v0, rows_v1)
        brow_v = (brow_v0, brow_v1)
        pltpu.sync_copy(words_hbm.at[pl.ds(base, b_per_w)], idx_v)

        def fire(c, slot):
            idx_c = idx_v.at[pl.ds(c * ch, ch)]
            pltpu.async_copy(wt_hbm.at[idx_c], rows_v[slot], sem_r)
            pltpu.async_copy(b_hbm.at[idx_c], brow_v[slot], sem_b)

        def drain(c, slot):
            pltpu.make_async_copy(
                wt_hbm.at[idx_v.at[pl.ds(c * ch, ch)]], rows_v[slot],
                sem_r).wait()
            pltpu.make_async_copy(
                b_hbm.at[idx_v.at[pl.ds(c * ch, ch)]], brow_v[slot],
                sem_b).wait()
            pltpu.sync_copy(rows_v[slot],
                            g_hbm.at[pl.ds(base + c * ch, ch)])
            pltpu.sync_copy(brow_v[slot],
                            bv_hbm.at[pl.ds(base + c * ch, ch)])

        fire(0, 0)
        for c in range(nchunk):
            if c + 1 < nchunk:
                fire(c + 1, (c + 1) % 2)
            drain(c, c % 2)

    return k(wt, emit_b, words_flat)


def _log_softmax_rows(x):
    m = jnp.max(x, axis=1, keepdims=True)
    return x - m - jnp.log(jnp.sum(jnp.exp(x - m), axis=1, keepdims=True))


def _emit_body(g_ref, bv_ref, predb_ref, pred_ref, lse_ref,
               root_W_ref, root_b_ref, rule_W_ref, rule_b_ref,
               op_W_ref, op_b_ref, s_in_W_ref, s_in_b_ref,
               r1_W1_ref, r1_b1_ref, r1_W2_ref, r1_b2_ref,
               r2_W1_ref, r2_b1_ref, r2_W2_ref, r2_b2_ref,
               s_out_W_ref, s_out_b_ref,
               x_ref, root_ref, rule_ref, op_ref, split_ref):
    predb = predb_ref[...]
    lse = lse_ref[...]
    for p in range(LB):
        x = lax.dot_general(predb,
                            g_ref[p * B:(p + 1) * B, :].astype(jnp.bfloat16),
                            (((1,), (1,)), ((), ())),
                            preferred_element_type=jnp.float32)
        x = x + bv_ref[p].reshape(1, B) - lse
        x_ref[p] = x

    @pl.when(pl.program_id(0) == 0)
    def _():
        root_ref[...] = _log_softmax_rows(root_W_ref[...] + root_b_ref[...])
        rule_ref[...] = _log_softmax_rows(rule_W_ref[...] + rule_b_ref[...])
        op_ref[...] = _log_softmax_rows(op_W_ref[...] + op_b_ref[...]).T
        pred = pred_ref[...]
        h = jnp.dot(pred, s_in_W_ref[...],
                    preferred_element_type=jnp.float32) + s_in_b_ref[...]
        t = jax.nn.relu(jnp.dot(h, r1_W1_ref[...],
                                preferred_element_type=jnp.float32)
                        + r1_b1_ref[...])
        h = h + jax.nn.relu(jnp.dot(t, r1_W2_ref[...],
                                    preferred_element_type=jnp.float32)
                            + r1_b2_ref[...])
        t = jax.nn.relu(jnp.dot(h, r2_W1_ref[...],
                                preferred_element_type=jnp.float32)
                        + r2_b1_ref[...])
        h = h + jax.nn.relu(jnp.dot(t, r2_W2_ref[...],
                                    preferred_element_type=jnp.float32)
                            + r2_b2_ref[...])
        sp = jnp.dot(h, s_out_W_ref[...],
                     preferred_element_type=jnp.float32) + s_out_b_ref[...]
        split_ref[...] = _log_softmax_rows(sp).T


def _emit(g, bv2, predb, predcat_emb, lse, root_W, root_b2, rule_W, rule_b2,
          op_W, op_b2, s_in_W, s_in_b2, r1_W1, r1_b12, r1_W2, r1_b22,
          r2_W1, r2_b12, r2_W2, r2_b22, s_out_W, s_out_b2):
    full = lambda shape: pl.BlockSpec(shape, lambda i: (0,) * len(shape))
    return pl.pallas_call(
        _emit_body,
        grid=(NLB,),
        in_specs=[
            pl.BlockSpec((LB * B, STATE), lambda i: (i, 0)),
            pl.BlockSpec((LB, 1, B), lambda i: (i, 0, 0)),
            full((QALL, STATE)),
            full((QALL, STATE)),
            full((QALL, 1)),
            full((1, QALL)), full((1, QALL)),
            full(rule_W.shape), full((1, rule_W.shape[1])),
            full(op_W.shape), full((1, op_W.shape[1])),
            full(s_in_W.shape), full((1, STATE)),
            full(r1_W1.shape), full((1, STATE)),
            full(r1_W2.shape), full((1, STATE)),
            full(r2_W1.shape), full((1, STATE)),
            full(r2_W2.shape), full((1, STATE)),
            full(s_out_W.shape), full((1, s_out_W.shape[1])),
        ],
        out_specs=[
            pl.BlockSpec((LB, QALL, B), lambda i: (i, 0, 0)),
            full((1, QALL)),
            full(rule_W.shape),
            full(op_W.shape[::-1]),
            full((s_out_W.shape[1], QALL)),
        ],
        out_shape=[
            jax.ShapeDtypeStruct((L, QALL, B), jnp.float32),
            jax.ShapeDtypeStruct((1, QALL), jnp.float32),
            jax.ShapeDtypeStruct(rule_W.shape, jnp.float32),
            jax.ShapeDtypeStruct(op_W.shape[::-1], jnp.float32),
            jax.ShapeDtypeStruct((s_out_W.shape[1], QALL), jnp.float32),
        ],
    )(g, bv2, predb, predcat_emb, lse, root_W, root_b2, rule_W, rule_b2,
      op_W, op_b2, s_in_W, s_in_b2, r1_W1, r1_b12, r1_W2, r1_b22,
      r2_W1, r2_b12, r2_W2, r2_b22, s_out_W, s_out_b2)


def kernel(words, emit_W, emit_b, predcat_emb, root_W, root_b, rule_W, rule_b,
           op_W, op_b, s_in_W, s_in_b, r1_W1, r1_b1, r1_W2, r1_b2,
           r2_W1, r2_b1, r2_W2, r2_b2, s_out_W, s_out_b):
    emit_wt = emit_W.T
    lse = _lse(predcat_emb, emit_wt, emit_b.reshape(NCHUNK, 1, VC))
    words_flat = words.T.reshape(NWORDS).astype(jnp.int32)
    g, bv = _sc_gather(emit_wt, emit_b, words_flat)
    x_p, root, rule, op, split = _emit(
        g, bv.reshape(L, 1, B), predcat_emb.astype(jnp.bfloat16),
        predcat_emb, lse,
        root_W, root_b.reshape(1, QALL), rule_W, rule_b.reshape(1, -1),
        op_W, op_b.reshape(1, -1), s_in_W, s_in_b.reshape(1, -1),
        r1_W1, r1_b1.reshape(1, -1), r1_W2, r1_b2.reshape(1, -1),
        r2_W1, r2_b1.reshape(1, -1), r2_W2, r2_b2.reshape(1, -1),
        s_out_W, s_out_b.reshape(1, -1))
    x = jnp.transpose(x_p, (2, 0, 1))
    return (x, root.reshape(QALL), rule, op.T, split.T)


# speedup vs baseline: 1.5619x; 1.0180x over previous
import functools

import jax
import jax.numpy as jnp
from jax import lax
from jax.experimental import pallas as pl
from jax.experimental.pallas import tpu as pltpu
from jax.experimental.pallas import tpu_sc as plsc

STATE = 128
VOCAB = 100000
QALL = 300
B = 1024
L = 50
NWORDS = B * L

VC = 10000
NCHUNK = VOCAB // VC
LB = 10
NLB = L // LB

_NEG = -1e30


def _lse_body(pred_ref, wt_ref, b_ref, lse_ref, m_ref, s_ref):
    i = pl.program_id(0)
    logits = lax.dot_general(pred_ref[...], wt_ref[...],
                             (((1,), (1,)), ((), ())),
                             preferred_element_type=jnp.float32)
    logits = logits + b_ref[...].reshape(1, VC)

    @pl.when(i == 0)
    def _():
        m_ref[...] = jnp.full((QALL, 1), _NEG, jnp.float32)
        s_ref[...] = jnp.zeros((QALL, 1), jnp.float32)

    def update(lm):
        m_old = m_ref[...]
        s_old = s_ref[...]
        m_new = jnp.maximum(m_old, jnp.max(lm, axis=1, keepdims=True))
        s_new = s_old * jnp.exp(m_old - m_new) + jnp.sum(
            jnp.exp(lm - m_new), axis=1, keepdims=True)
        m_ref[...] = m_new
        s_ref[...] = s_new
        return m_new, s_new

    m_new, s_new = update(logits)

    @pl.when(i == NCHUNK - 1)
    def _():
        lse_ref[...] = m_new + jnp.log(s_new)


def _lse(predcat_emb, emit_wt, emit_b2):
    return pl.pallas_call(
        _lse_body,
        grid=(NCHUNK,),
        in_specs=[
            pl.BlockSpec((QALL, STATE), lambda i: (0, 0)),
            pl.BlockSpec((VC, STATE), lambda i: (i, 0)),
            pl.BlockSpec((1, 1, VC), lambda i: (i, 0, 0)),
        ],
        out_specs=pl.BlockSpec((QALL, 1), lambda i: (0, 0)),
        out_shape=jax.ShapeDtypeStruct((QALL, 1), jnp.float32),
        scratch_shapes=[
            pltpu.VMEM((QALL, 1), jnp.float32),
            pltpu.VMEM((QALL, 1), jnp.float32),
        ],
    )(predcat_emb, emit_wt, emit_b2)


def _sc_gather(wt, emit_b, words_flat):
    info = plsc.get_sparse_core_info()
    nc, ns = info.num_cores, info.num_subcores
    nw = nc * ns
    b_per_w = NWORDS // nw
    nchunk = 4
    ch = b_per_w // nchunk

    mesh = plsc.VectorSubcoreMesh(core_axis_name="c", subcore_axis_name="s")

    @functools.partial(
        pl.kernel, mesh=mesh,
        out_type=[
            jax.ShapeDtypeStruct((NWORDS, STATE), jnp.float32),
            jax.ShapeDtypeStruct((NWORDS,), jnp.float32),
        ],
        scratch_types=[
            pltpu.VMEM((b_per_w,), jnp.int32),
            pltpu.VMEM((ch, STATE), jnp.float32),
            pltpu.VMEM((ch, STATE), jnp.float32),
            pltpu.VMEM((ch,), jnp.float32),
            pltpu.VMEM((ch,), jnp.float32),
            pltpu.SemaphoreType.DMA,
            pltpu.SemaphoreType.DMA,
        ],
    )
    def k(wt_hbm, b_hbm, words_hbm, g_hbm, bv_hbm, idx_v, rows_v0, rows_v1,
          brow_v0, brow_v1, sem_r, sem_b):
        wid = lax.axis_index("s") * nc + lax.axis_index("c")
        base = wid * b_per_w
        rows_v = (rows_v0, rows_v1)
        brow_v = (brow_v0, brow_v1)
        pltpu.sync_copy(words_hbm.at[pl.ds(base, b_per_w)], idx_v)

        def fire(c, slot):
            idx_c = idx_v.at[pl.ds(c * ch, ch)]
            pltpu.async_copy(wt_hbm.at[idx_c], rows_v[slot], sem_r)
            pltpu.async_copy(b_hbm.at[idx_c], brow_v[slot], sem_b)

        def drain(c, slot):
            pltpu.make_async_copy(
                wt_hbm.at[idx_v.at[pl.ds(c * ch, ch)]], rows_v[slot],
                sem_r).wait()
            pltpu.make_async_copy(
                b_hbm.at[idx_v.at[pl.ds(c * ch, ch)]], brow_v[slot],
                sem_b).wait()
            pltpu.sync_copy(rows_v[slot],
                            g_hbm.at[pl.ds(base + c * ch, ch)])
            pltpu.sync_copy(brow_v[slot],
                            bv_hbm.at[pl.ds(base + c * ch, ch)])

        fire(0, 0)
        for c in range(nchunk):
            if c + 1 < nchunk:
                fire(c + 1, (c + 1) % 2)
            drain(c, c % 2)

    return k(wt, emit_b, words_flat)


def _log_softmax_rows(x):
    m = jnp.max(x, axis=1, keepdims=True)
    return x - m - jnp.log(jnp.sum(jnp.exp(x - m), axis=1, keepdims=True))


def _emit_body(g_ref, bv_ref, predb_ref, pred_ref, lse_ref,
               root_W_ref, root_b_ref, rule_W_ref, rule_b_ref,
               op_W_ref, op_b_ref, s_in_W_ref, s_in_b_ref,
               r1_W1_ref, r1_b1_ref, r1_W2_ref, r1_b2_ref,
               r2_W1_ref, r2_b1_ref, r2_W2_ref, r2_b2_ref,
               s_out_W_ref, s_out_b_ref,
               x_ref, root_ref, rule_ref, op_ref, split_ref):
    predb = predb_ref[...]
    lse = lse_ref[...]
    for p in range(LB):
        x = lax.dot_general(predb,
                            g_ref[p * B:(p + 1) * B, :].astype(jnp.bfloat16),
                            (((1,), (1,)), ((), ())),
                            preferred_element_type=jnp.float32)
        x = x + bv_ref[p].reshape(1, B) - lse
        x_ref[p] = x

    @pl.when(pl.program_id(0) == 0)
    def _():
        root_ref[...] = _log_softmax_rows(root_W_ref[...] + root_b_ref[...])
        rule_ref[...] = _log_softmax_rows(rule_W_ref[...] + rule_b_ref[...])
        op_ref[...] = _log_softmax_rows(op_W_ref[...] + op_b_ref[...]).T
        pred = pred_ref[...]
        h = jnp.dot(pred, s_in_W_ref[...],
                    preferred_element_type=jnp.float32) + s_in_b_ref[...]
        t = jax.nn.relu(jnp.dot(h, r1_W1_ref[...],
                                preferred_element_type=jnp.float32)
                        + r1_b1_ref[...])
        h = h + jax.nn.relu(jnp.dot(t, r1_W2_ref[...],
                                    preferred_element_type=jnp.float32)
                            + r1_b2_ref[...])
        t = jax.nn.relu(jnp.dot(h, r2_W1_ref[...],
                                preferred_element_type=jnp.float32)
                        + r2_b1_ref[...])
        h = h + jax.nn.relu(jnp.dot(t, r2_W2_ref[...],
                                    preferred_element_type=jnp.float32)
                            + r2_b2_ref[...])
        sp = jnp.dot(h, s_out_W_ref[...],
                     preferred_element_type=jnp.float32) + s_out_b_ref[...]
        split_ref[...] = _log_softmax_rows(sp).T


def _emit(g, bv2, predb, predcat_emb, lse, root_W, root_b2, rule_W, rule_b2,
          op_W, op_b2, s_in_W, s_in_b2, r1_W1, r1_b12, r1_W2, r1_b22,
          r2_W1, r2_b12, r2_W2, r2_b22, s_out_W, s_out_b2):
    full = lambda shape: pl.BlockSpec(shape, lambda i: (0,) * len(shape))
    return pl.pallas_call(
        _emit_body,
        grid=(NLB,),
        in_specs=[
            pl.BlockSpec((LB * B, STATE), lambda i: (i, 0)),
            pl.BlockSpec((LB, 1, B), lambda i: (i, 0, 0)),
            full((QALL, STATE)),
            full((QALL, STATE)),
            full((QALL, 1)),
            full((1, QALL)), full((1, QALL)),
            full(rule_W.shape), full((1, rule_W.shape[1])),
            full(op_W.shape), full((1, op_W.shape[1])),
            full(s_in_W.shape), full((1, STATE)),
            full(r1_W1.shape), full((1, STATE)),
            full(r1_W2.shape), full((1, STATE)),
            full(r2_W1.shape), full((1, STATE)),
            full(r2_W2.shape), full((1, STATE)),
            full(s_out_W.shape), full((1, s_out_W.shape[1])),
        ],
        out_specs=[
            pl.BlockSpec((LB, QALL, B), lambda i: (i, 0, 0)),
            full((1, QALL)),
            full(rule_W.shape),
            full(op_W.shape[::-1]),
            full((s_out_W.shape[1], QALL)),
        ],
        out_shape=[
            jax.ShapeDtypeStruct((L, QALL, B), jnp.float32),
            jax.ShapeDtypeStruct((1, QALL), jnp.float32),
            jax.ShapeDtypeStruct(rule_W.shape, jnp.float32),
            jax.ShapeDtypeStruct(op_W.shape[::-1], jnp.float32),
            jax.ShapeDtypeStruct((s_out_W.shape[1], QALL), jnp.float32),
        ],
    )(g, bv2, predb, predcat_emb, lse, root_W, root_b2, rule_W, rule_b2,
      op_W, op_b2, s_in_W, s_in_b2, r1_W1, r1_b12, r1_W2, r1_b22,
      r2_W1, r2_b12, r2_W2, r2_b22, s_out_W, s_out_b2)


def kernel(words, emit_W, emit_b, predcat_emb, root_W, root_b, rule_W, rule_b,
           op_W, op_b, s_in_W, s_in_b, r1_W1, r1_b1, r1_W2, r1_b2,
           r2_W1, r2_b1, r2_W2, r2_b2, s_out_W, s_out_b):
    emit_wt = emit_W.T
    lse = _lse(predcat_emb, emit_wt, emit_b.reshape(NCHUNK, 1, VC))
    words_flat = words.T.reshape(NWORDS).astype(jnp.int32)
    g, bv = _sc_gather(emit_wt, emit_b, words_flat)
    x_p, root, rule, op, split = _emit(
        g, bv.reshape(L, 1, B), predcat_emb.astype(jnp.bfloat16),
        predcat_emb, lse,
        root_W, root_b.reshape(1, QALL), rule_W, rule_b.reshape(1, -1),
        op_W, op_b.reshape(1, -1), s_in_W, s_in_b.reshape(1, -1),
        r1_W1, r1_b1.reshape(1, -1), r1_W2, r1_b2.reshape(1, -1),
        r2_W1, r2_b1.reshape(1, -1), r2_W2, r2_b2.reshape(1, -1),
        s_out_W, s_out_b.reshape(1, -1))
    x = jnp.transpose(x_p, (2, 0, 1))
    return (x, root.reshape(QALL), rule, op.T, split.T)
